# baseline (speedup 1.0000x reference)
import functools

import jax
import jax.numpy as jnp
from jax import lax
from jax.experimental import pallas as pl
from jax.experimental.pallas import tpu as pltpu
from jax.experimental.pallas import tpu_sc as plsc

B = 16384
D = 128
NC = 2
NS = 16
NW = NC * NS
BPW = B // NW
NBLK = 4
BLK = BPW // NBLK
LANES = 16
NCH = D // LANES


@jax.jit
def _sc_embed(idx, table):
    mesh = plsc.VectorSubcoreMesh(core_axis_name="c", subcore_axis_name="s")

    @functools.partial(
        pl.kernel,
        mesh=mesh,
        out_type=jax.ShapeDtypeStruct((B, D), jnp.float32),
        scratch_types=[
            pltpu.VMEM((BPW,), jnp.int32),
            pltpu.VMEM((2 * D,), jnp.float32),
            pltpu.VMEM((BPW, D), jnp.float32),
            pltpu.SemaphoreType.DMA,
            pltpu.SemaphoreType.DMA,
        ],
    )
    def k(idx_ref, table_ref, out_ref, idx_v, tab_v, rows_v, sem, lsem):
        wid = lax.axis_index("s") * NC + lax.axis_index("c")
        ld_idx = pltpu.async_copy(idx_ref.at[wid], idx_v, lsem)
        ld_tab = pltpu.async_copy(table_ref, tab_v, lsem)
        ld_idx.wait()
        ld_tab.wait()
        w0 = [tab_v[pl.ds(c * LANES, LANES)] for c in range(NCH)]
        dif = [tab_v[pl.ds(D + c * LANES, LANES)] - w0[c] for c in range(NCH)]

        def body(g, carry):
            pv = idx_v[pl.ds(g * LANES, LANES)]
            for r in range(LANES):
                bc = pv.at[jnp.full((LANES,), r, jnp.int32)].get(
                    mode="promise_in_bounds")
                pf = bc.astype(jnp.float32)
                row = g * LANES + r
                for c in range(NCH):
                    rows_v[row, pl.ds(c * LANES, LANES)] = (
                        w0[c] + pf * dif[c])
            return carry

        grps = BLK // LANES
        copies = []
        for b in range(NBLK):
            lax.fori_loop(b * grps, (b + 1) * grps, body, 0)
            copies.append(pltpu.async_copy(
                rows_v.at[pl.ds(b * BLK, BLK)],
                out_ref.at[pl.ds(wid * BPW + b * BLK, BLK)],
                sem,
            ))
        for cp in copies:
            cp.wait()

    return k(idx, table)


def kernel(polarities, embedding_weight):
    idx = polarities.astype(jnp.int32).reshape(NW, BPW)
    return _sc_embed(idx, embedding_weight.reshape(2 * D))


# speedup vs baseline: 1.0321x; 1.0321x over previous
import functools

import jax
import jax.numpy as jnp
from jax import lax
from jax.experimental import pallas as pl
from jax.experimental.pallas import tpu as pltpu
from jax.experimental.pallas import tpu_sc as plsc

B = 16384
D = 128
NC = 2
NS = 16
NW = NC * NS
BPW = B // NW
NBLK = 2
BLK = BPW // NBLK
LANES = 16
NCH = D // LANES


@jax.jit
def _sc_embed(idx, table):
    mesh = plsc.VectorSubcoreMesh(core_axis_name="c", subcore_axis_name="s")

    @functools.partial(
        pl.kernel,
        mesh=mesh,
        out_type=jax.ShapeDtypeStruct((B, D), jnp.float32),
        scratch_types=[
            pltpu.VMEM((BPW,), jnp.int32),
            pltpu.VMEM((2, D), jnp.float32),
            pltpu.VMEM((BPW, D), jnp.float32),
            pltpu.SemaphoreType.DMA,
            pltpu.SemaphoreType.DMA,
        ],
    )
    def k(idx_ref, table_ref, out_ref, idx_v, tab_v, rows_v, sem, lsem):
        wid = lax.axis_index("s") * NC + lax.axis_index("c")
        ld_idx = pltpu.async_copy(idx_ref.at[wid], idx_v, lsem)
        ld_tab = pltpu.async_copy(table_ref, tab_v, lsem)
        ld_idx.wait()
        ld_tab.wait()
        w0 = [tab_v[0, pl.ds(c * LANES, LANES)] for c in range(NCH)]
        dif = [tab_v[1, pl.ds(c * LANES, LANES)] - w0[c] for c in range(NCH)]

        def body(g, carry):
            pv = idx_v[pl.ds(g * LANES, LANES)]
            for r in range(LANES):
                bc = pv.at[jnp.full((LANES,), r, jnp.int32)].get(
                    mode="promise_in_bounds")
                pf = bc.astype(jnp.float32)
                row = g * LANES + r
                for c in range(NCH):
                    rows_v[row, pl.ds(c * LANES, LANES)] = (
                        w0[c] + pf * dif[c])
            return carry

        grps = BLK // LANES
        copies = []
        for b in range(NBLK):
            lax.fori_loop(b * grps, (b + 1) * grps, body, 0)
            copies.append(pltpu.async_copy(
                rows_v.at[pl.ds(b * BLK, BLK)],
                out_ref.at[pl.ds(wid * BPW + b * BLK, BLK)],
                sem,
            ))
        for cp in copies:
            cp.wait()

    return k(idx, table)


def kernel(polarities, embedding_weight):
    idx = polarities.astype(jnp.int32).reshape(NW, BPW)
    return _sc_embed(idx, embedding_weight)
